# baseline (speedup 1.0000x reference)
import jax
import jax.numpy as jnp
from jax.experimental import pallas as pl


def _copy_body(w_ref, o_ref):
    o_ref[...] = w_ref[...]


def kernel(tokens, W_pos):
    batch, seq = tokens.shape
    d = W_pos.shape[1]
    blk = 1024
    nj = seq // blk
    out = pl.pallas_call(
        _copy_body,
        grid=(nj, batch),
        in_specs=[pl.BlockSpec((blk, d), lambda j, r: (j, 0))],
        out_specs=pl.BlockSpec((blk, d), lambda j, r: (r * nj + j, 0)),
        out_shape=jax.ShapeDtypeStruct((batch * seq, d), W_pos.dtype),
    )(W_pos)
    return out.reshape(batch, seq, d)
